# baseline (speedup 1.0000x reference)
import jax
import jax.numpy as jnp
from jax.experimental import pallas as pl
from jax.experimental.pallas import tpu as pltpu

_N_BINS = 15
_ROWS, _COLS = 16384, 2048
_RING = 4
_BUF_ROWS = 1024
_CHUNKS = [256, 512, 512] + [1024] * 14 + [512, 256]
assert sum(_CHUNKS) == _ROWS
_OFFS = [sum(_CHUNKS[:k]) for k in range(len(_CHUNKS))]


def _ring_kernel(temp_ref, bc_ref, bt_ref, x_hbm,
                 xout_hbm, ece_ref, tout_ref, acc_ref,
                 b0, b1, b2, b3, insem, outsem):
    bufs = [b0, b1, b2, b3]

    def in_cp(k):
        b = k % _RING
        return pltpu.make_async_copy(
            x_hbm.at[pl.ds(_OFFS[k], _CHUNKS[k])],
            bufs[b].at[pl.ds(0, _CHUNKS[k])],
            insem.at[b])

    def out_cp(k):
        b = k % _RING
        return pltpu.make_async_copy(
            bufs[b].at[pl.ds(0, _CHUNKS[k])],
            xout_hbm.at[pl.ds(_OFFS[k], _CHUNKS[k])],
            outsem.at[b])

    for k in range(_RING):
        in_cp(k).start()

    n = jnp.float32(0.0)
    for i in range(_N_BINS):
        n = n + bt_ref[i]
    s = jnp.float32(0.0)
    for i in range(_N_BINS):
        bc = bc_ref[i]
        bt = bt_ref[i]
        acc = bc / (bt + 1e-8)
        acc_ref[i] = acc
        conf = i / (_N_BINS - 1.0) + 0.5 / _N_BINS
        s = s + bt * jnp.abs(acc - conf)
    ece_ref[0] = jnp.where(n > 0.0, s / jnp.maximum(n, 1e-8), 0.0)
    tout_ref[0] = jnp.clip(temp_ref[0], 0.1, 10.0)

    nk = len(_CHUNKS)
    for k in range(nk):
        in_cp(k).wait()
        out_cp(k).start()
        if k + _RING < nk:
            out_cp(k).wait()
            in_cp(k + _RING).start()
    for k in range(max(nk - _RING, 0), nk):
        out_cp(k).wait()


def kernel(x, temperature, platt_a, platt_b, bin_correct, bin_total):
    xout, ece, temp, acc = pl.pallas_call(
        _ring_kernel,
        out_shape=(
            jax.ShapeDtypeStruct((_ROWS, _COLS), jnp.float32),
            jax.ShapeDtypeStruct((1,), jnp.float32),
            jax.ShapeDtypeStruct((1,), jnp.float32),
            jax.ShapeDtypeStruct((_N_BINS,), jnp.float32),
        ),
        in_specs=[
            pl.BlockSpec(memory_space=pltpu.SMEM),
            pl.BlockSpec(memory_space=pltpu.SMEM),
            pl.BlockSpec(memory_space=pltpu.SMEM),
            pl.BlockSpec(memory_space=pl.ANY),
        ],
        out_specs=(
            pl.BlockSpec(memory_space=pl.ANY),
            pl.BlockSpec(memory_space=pltpu.SMEM),
            pl.BlockSpec(memory_space=pltpu.SMEM),
            pl.BlockSpec(memory_space=pltpu.SMEM),
        ),
        scratch_shapes=[
            pltpu.VMEM((_BUF_ROWS, _COLS), jnp.float32),
            pltpu.VMEM((_BUF_ROWS, _COLS), jnp.float32),
            pltpu.VMEM((_BUF_ROWS, _COLS), jnp.float32),
            pltpu.VMEM((_BUF_ROWS, _COLS), jnp.float32),
            pltpu.SemaphoreType.DMA((_RING,)),
            pltpu.SemaphoreType.DMA((_RING,)),
        ],
    )(temperature.reshape(1), bin_correct, bin_total, x)
    return (xout, ece.reshape(()), temp.reshape(()), acc)


# speedup vs baseline: 1.0124x; 1.0124x over previous
import jax
import jax.numpy as jnp
from jax.experimental import pallas as pl
from jax.experimental.pallas import tpu as pltpu

_N_BINS = 15
_ROWS, _COLS = 16384, 2048
_RING = 6
_DEPTH = 4
_BUF_ROWS = 1024
_CHUNKS = [256, 512, 512] + [1024] * 14 + [512, 256]
assert sum(_CHUNKS) == _ROWS
_OFFS = [sum(_CHUNKS[:k]) for k in range(len(_CHUNKS))]


def _ring_kernel(temp_ref, bc_ref, bt_ref, x_hbm,
                 xout_hbm, ece_ref, tout_ref, acc_ref,
                 b0, b1, b2, b3, b4, b5, insem, outsem):
    bufs = [b0, b1, b2, b3, b4, b5]

    def in_cp(k):
        b = k % _RING
        return pltpu.make_async_copy(
            x_hbm.at[pl.ds(_OFFS[k], _CHUNKS[k])],
            bufs[b].at[pl.ds(0, _CHUNKS[k])],
            insem.at[b])

    def out_cp(k):
        b = k % _RING
        return pltpu.make_async_copy(
            bufs[b].at[pl.ds(0, _CHUNKS[k])],
            xout_hbm.at[pl.ds(_OFFS[k], _CHUNKS[k])],
            outsem.at[b])

    for k in range(_DEPTH):
        in_cp(k).start()

    n = jnp.float32(0.0)
    for i in range(_N_BINS):
        n = n + bt_ref[i]
    s = jnp.float32(0.0)
    for i in range(_N_BINS):
        bc = bc_ref[i]
        bt = bt_ref[i]
        acc = bc / (bt + 1e-8)
        acc_ref[i] = acc
        conf = i / (_N_BINS - 1.0) + 0.5 / _N_BINS
        s = s + bt * jnp.abs(acc - conf)
    ece_ref[0] = jnp.where(n > 0.0, s / jnp.maximum(n, 1e-8), 0.0)
    tout_ref[0] = jnp.clip(temp_ref[0], 0.1, 10.0)

    nk = len(_CHUNKS)
    waited = [False] * nk
    for k in range(nk):
        in_cp(k).wait()
        out_cp(k).start()
        j = k + _DEPTH
        if j < nk:
            if j >= _RING:
                out_cp(j - _RING).wait()
                waited[j - _RING] = True
            in_cp(j).start()
    for k in range(nk):
        if not waited[k]:
            out_cp(k).wait()


def kernel(x, temperature, platt_a, platt_b, bin_correct, bin_total):
    xout, ece, temp, acc = pl.pallas_call(
        _ring_kernel,
        out_shape=(
            jax.ShapeDtypeStruct((_ROWS, _COLS), jnp.float32),
            jax.ShapeDtypeStruct((1,), jnp.float32),
            jax.ShapeDtypeStruct((1,), jnp.float32),
            jax.ShapeDtypeStruct((_N_BINS,), jnp.float32),
        ),
        in_specs=[
            pl.BlockSpec(memory_space=pltpu.SMEM),
            pl.BlockSpec(memory_space=pltpu.SMEM),
            pl.BlockSpec(memory_space=pltpu.SMEM),
            pl.BlockSpec(memory_space=pl.ANY),
        ],
        out_specs=(
            pl.BlockSpec(memory_space=pl.ANY),
            pl.BlockSpec(memory_space=pltpu.SMEM),
            pl.BlockSpec(memory_space=pltpu.SMEM),
            pl.BlockSpec(memory_space=pltpu.SMEM),
        ),
        scratch_shapes=[
            pltpu.VMEM((_BUF_ROWS, _COLS), jnp.float32),
            pltpu.VMEM((_BUF_ROWS, _COLS), jnp.float32),
            pltpu.VMEM((_BUF_ROWS, _COLS), jnp.float32),
            pltpu.VMEM((_BUF_ROWS, _COLS), jnp.float32),
            pltpu.VMEM((_BUF_ROWS, _COLS), jnp.float32),
            pltpu.VMEM((_BUF_ROWS, _COLS), jnp.float32),
            pltpu.SemaphoreType.DMA((_RING,)),
            pltpu.SemaphoreType.DMA((_RING,)),
        ],
    )(temperature.reshape(1), bin_correct, bin_total, x)
    return (xout, ece.reshape(()), temp.reshape(()), acc)


# speedup vs baseline: 1.0199x; 1.0074x over previous
import jax
import jax.numpy as jnp
from jax.experimental import pallas as pl
from jax.experimental.pallas import tpu as pltpu

_N_BINS = 15
_ROWS, _COLS = 16384, 2048
_BLK = 1024


def _fused_kernel(temp_ref, bc_ref, bt_ref, x_ref,
                  xout_ref, ece_ref, tout_ref, acc_ref):
    xout_ref[...] = x_ref[...]

    @pl.when(pl.program_id(0) == 0)
    def _stats():
        n = jnp.float32(0.0)
        for i in range(_N_BINS):
            n = n + bt_ref[i]
        s = jnp.float32(0.0)
        for i in range(_N_BINS):
            bc = bc_ref[i]
            bt = bt_ref[i]
            acc = bc / (bt + 1e-8)
            acc_ref[i] = acc
            conf = i / (_N_BINS - 1.0) + 0.5 / _N_BINS
            s = s + bt * jnp.abs(acc - conf)
        ece_ref[0] = jnp.where(n > 0.0, s / jnp.maximum(n, 1e-8), 0.0)
        tout_ref[0] = jnp.clip(temp_ref[0], 0.1, 10.0)


def kernel(x, temperature, platt_a, platt_b, bin_correct, bin_total):
    xout, ece, temp, acc = pl.pallas_call(
        _fused_kernel,
        grid=(_ROWS // _BLK,),
        out_shape=(
            jax.ShapeDtypeStruct((_ROWS, _COLS), jnp.float32),
            jax.ShapeDtypeStruct((1,), jnp.float32),
            jax.ShapeDtypeStruct((1,), jnp.float32),
            jax.ShapeDtypeStruct((_N_BINS,), jnp.float32),
        ),
        in_specs=[
            pl.BlockSpec(memory_space=pltpu.SMEM),
            pl.BlockSpec(memory_space=pltpu.SMEM),
            pl.BlockSpec(memory_space=pltpu.SMEM),
            pl.BlockSpec((_BLK, _COLS), lambda i: (i, 0),
                         pipeline_mode=pl.Buffered(buffer_count=2)),
        ],
        out_specs=(
            pl.BlockSpec((_BLK, _COLS), lambda i: (i, 0),
                         pipeline_mode=pl.Buffered(buffer_count=2)),
            pl.BlockSpec(memory_space=pltpu.SMEM),
            pl.BlockSpec(memory_space=pltpu.SMEM),
            pl.BlockSpec(memory_space=pltpu.SMEM),
        ),
        compiler_params=pltpu.CompilerParams(vmem_limit_bytes=128 * 1024 * 1024),
    )(temperature.reshape(1), bin_correct, bin_total, x)
    return (xout, ece.reshape(()), temp.reshape(()), acc)


# speedup vs baseline: 1.0208x; 1.0008x over previous
import jax
import jax.numpy as jnp
from jax.experimental import pallas as pl
from jax.experimental.pallas import tpu as pltpu

_N_BINS = 15
_ROWS, _COLS = 16384, 2048
_BLK = 1024


def _fused_kernel(temp_ref, bc_ref, bt_ref, x_ref,
                  xout_ref, ece_ref, tout_ref, acc_ref):
    xout_ref[...] = x_ref[...]

    @pl.when(pl.program_id(0) == 7)
    def _stats():
        n = jnp.float32(0.0)
        for i in range(_N_BINS):
            n = n + bt_ref[i]
        s = jnp.float32(0.0)
        for i in range(_N_BINS):
            bc = bc_ref[i]
            bt = bt_ref[i]
            acc = bc / (bt + 1e-8)
            acc_ref[i] = acc
            conf = i / (_N_BINS - 1.0) + 0.5 / _N_BINS
            s = s + bt * jnp.abs(acc - conf)
        ece_ref[0] = jnp.where(n > 0.0, s / jnp.maximum(n, 1e-8), 0.0)
        tout_ref[0] = jnp.clip(temp_ref[0], 0.1, 10.0)


def kernel(x, temperature, platt_a, platt_b, bin_correct, bin_total):
    xout, ece, temp, acc = pl.pallas_call(
        _fused_kernel,
        grid=(_ROWS // _BLK,),
        out_shape=(
            jax.ShapeDtypeStruct((_ROWS, _COLS), jnp.float32),
            jax.ShapeDtypeStruct((1,), jnp.float32),
            jax.ShapeDtypeStruct((1,), jnp.float32),
            jax.ShapeDtypeStruct((_N_BINS,), jnp.float32),
        ),
        in_specs=[
            pl.BlockSpec(memory_space=pltpu.SMEM),
            pl.BlockSpec(memory_space=pltpu.SMEM),
            pl.BlockSpec(memory_space=pltpu.SMEM),
            pl.BlockSpec((_BLK, _COLS), lambda i: (i, 0),
                         pipeline_mode=pl.Buffered(buffer_count=2)),
        ],
        out_specs=(
            pl.BlockSpec((_BLK, _COLS), lambda i: (i, 0),
                         pipeline_mode=pl.Buffered(buffer_count=2)),
            pl.BlockSpec(memory_space=pltpu.SMEM),
            pl.BlockSpec(memory_space=pltpu.SMEM),
            pl.BlockSpec(memory_space=pltpu.SMEM),
        ),
        compiler_params=pltpu.CompilerParams(vmem_limit_bytes=128 * 1024 * 1024),
    )(temperature.reshape(1), bin_correct, bin_total, x)
    return (xout, ece.reshape(()), temp.reshape(()), acc)
